# baseline (speedup 1.0000x reference)
import functools

import jax
import jax.numpy as jnp
from jax import lax
from jax.experimental import pallas as pl
from jax.experimental.pallas import tpu as pltpu
from jax.experimental.pallas import tpu_sc as plsc

BATCH = 16384
FIELDS = 26
EMBED_DIM = 16

V = 1000001
NBLK = 7813
VP = NBLK * 128
NUM_CORES = 2
NUM_SUBCORES = 16
NW = NUM_CORES * NUM_SUBCORES

B = BATCH * FIELDS
BLK_PER_W = (NBLK + NW - 1) // NW

K2_CHUNK = 1024
K2_UNITS_PER_W = (FIELDS * (BATCH // K2_CHUNK)) // NW

K3_TILES = FIELDS * 2 * (BATCH // 128)
K3_PER_W = K3_TILES // NW

OFLAT = FIELDS * EMBED_DIM * BATCH


def _mesh():
    return plsc.VectorSubcoreMesh(core_axis_name="c", subcore_axis_name="s",
                                  num_cores=NUM_CORES)


@functools.lru_cache(maxsize=None)
def _build_k1():
    @functools.partial(
        pl.kernel,
        mesh=_mesh(),
        out_type=(
            jax.ShapeDtypeStruct((VP * EMBED_DIM,), jnp.float32),
            jax.ShapeDtypeStruct((B,), jnp.int32),
        ),
        scratch_types=[
            pltpu.VMEM((EMBED_DIM, 128), jnp.float32),
            pltpu.VMEM((128 * EMBED_DIM,), jnp.float32),
            pltpu.VMEM((8, 2048), jnp.int32),
        ],
        compiler_params=pltpu.CompilerParams(use_tc_tiling_on_sc=True, needs_layout_passes=False, disable_bounds_checks=True),
    )
    def k1(table_t, idx_t, tflat, iflat, blk_in, blk_out, ibuf):
        wid = lax.axis_index("s") * NUM_CORES + lax.axis_index("c")
        lanes = lax.iota(jnp.int32, 16)

        def blk_body(j, _):
            blk = wid * BLK_PER_W + j

            @pl.when(blk < NBLK)
            def _():
                v0 = pl.multiple_of(blk * 128, 128)
                pltpu.sync_copy(table_t.at[:, pl.ds(v0, 128)], blk_in)

                def v_body(v, _):
                    row = plsc.load_gather(blk_in, [lanes, lanes * 0 + v])
                    blk_out[pl.ds(v * EMBED_DIM, EMBED_DIM)] = row
                    return 0

                lax.fori_loop(0, 128, v_body, 0)
                pltpu.sync_copy(
                    blk_out, tflat.at[pl.ds(v0 * EMBED_DIM, 128 * EMBED_DIM)])

            return 0

        lax.fori_loop(0, BLK_PER_W, blk_body, 0)

        fblk = wid // 8
        iblk = wid % 8
        f0 = pl.multiple_of(fblk * 8, 8)
        i0 = pl.multiple_of(iblk * 2048, 128)
        pltpu.sync_copy(idx_t.at[pl.ds(f0, 8), pl.ds(i0, 2048)], ibuf)
        for r in range(8):
            @pl.when(f0 + r < FIELDS)
            def _():
                pltpu.sync_copy(
                    ibuf.at[r],
                    iflat.at[pl.ds((f0 + r) * BATCH + i0, 2048)])

    return k1


@functools.lru_cache(maxsize=None)
def _build_k2():
    @functools.partial(
        pl.kernel,
        mesh=_mesh(),
        out_type=jax.ShapeDtypeStruct((OFLAT,), jnp.float32),
        scratch_types=[
            pltpu.VMEM((K2_CHUNK,), jnp.int32),
            pltpu.VMEM((K2_CHUNK, EMBED_DIM), jnp.float32),
            pltpu.VMEM((EMBED_DIM * K2_CHUNK,), jnp.float32),
            pltpu.SemaphoreType.DMA,
        ],
        compiler_params=pltpu.CompilerParams(use_tc_tiling_on_sc=False,
                                             needs_layout_passes=False,
                                             disable_bounds_checks=True),
    )
    def k2(tflat2d, iflat, oflat, idxbuf, rows, tr, sem):
        wid = lax.axis_index("s") * NUM_CORES + lax.axis_index("c")
        lanes = lax.iota(jnp.int32, 16)

        def unit_body(j, _):
            u = wid * K2_UNITS_PER_W + j
            f = u // (BATCH // K2_CHUNK)
            c = u % (BATCH // K2_CHUNK)
            i0 = pl.multiple_of(c * K2_CHUNK, K2_CHUNK)
            pltpu.sync_copy(iflat.at[pl.ds(f * BATCH + i0, K2_CHUNK)], idxbuf)
            pltpu.async_copy(tflat2d.at[idxbuf], rows, sem).wait()

            def g_body(g, _):
                for d in range(EMBED_DIM):
                    vec = plsc.load_gather(rows, [lanes + g * 16, lanes * 0 + d])
                    tr[pl.ds(d * K2_CHUNK + g * 16, 16)] = vec
                return 0

            lax.fori_loop(0, K2_CHUNK // 16, g_body, 0)
            for d in range(EMBED_DIM):
                pltpu.sync_copy(
                    tr.at[pl.ds(d * K2_CHUNK, K2_CHUNK)],
                    oflat.at[pl.ds((f * EMBED_DIM + d) * BATCH + i0, K2_CHUNK)])
            return 0

        lax.fori_loop(0, K2_UNITS_PER_W, unit_body, 0)

    return k2


@functools.lru_cache(maxsize=None)
def _build_k3():
    @functools.partial(
        pl.kernel,
        mesh=_mesh(),
        out_type=jax.ShapeDtypeStruct((FIELDS, EMBED_DIM, BATCH), jnp.float32),
        scratch_types=[
            pltpu.VMEM((8, 128), jnp.float32),
        ],
        compiler_params=pltpu.CompilerParams(use_tc_tiling_on_sc=True, needs_layout_passes=False, disable_bounds_checks=True),
    )
    def k3(oflat, o3, tbuf):
        wid = lax.axis_index("s") * NUM_CORES + lax.axis_index("c")

        def t_body(j, _):
            t = wid * K3_PER_W + j
            f = t // (2 * (BATCH // 128))
            rem = t % (2 * (BATCH // 128))
            d0 = pl.multiple_of((rem // (BATCH // 128)) * 8, 8)
            i0 = pl.multiple_of((rem % (BATCH // 128)) * 128, 128)
            for r in range(8):
                pltpu.sync_copy(
                    oflat.at[pl.ds((f * EMBED_DIM + d0 + r) * BATCH + i0, 128)],
                    tbuf.at[r])
            pltpu.sync_copy(tbuf, o3.at[f, pl.ds(d0, 8), pl.ds(i0, 128)])
            return 0

        lax.fori_loop(0, K3_PER_W, t_body, 0)

    return k3


def kernel(inputs, table):
    idx_t = inputs.astype(jnp.int32).T
    table_t = table.T
    tflat, iflat = _build_k1()(table_t, idx_t)
    oflat = _build_k2()(tflat.reshape(VP, EMBED_DIM), iflat)
    o3 = _build_k3()(oflat)
    return o3.transpose(2, 0, 1)


# speedup vs baseline: 2.5641x; 2.5641x over previous
import functools

import jax
import jax.numpy as jnp
from jax import lax
from jax.experimental import pallas as pl
from jax.experimental.pallas import tpu as pltpu
from jax.experimental.pallas import tpu_sc as plsc

BATCH = 16384
FIELDS = 26
EMBED_DIM = 16

V = 1000001
NBLK = 7813
VP = NBLK * 128
NUM_CORES = 2
NUM_SUBCORES = 16
NW = NUM_CORES * NUM_SUBCORES

B = BATCH * FIELDS

CH1 = 1024
NFULL1 = VP // CH1
TAIL = VP - NFULL1 * CH1
T_RD = V - NFULL1 * CH1

K2C = 1024

K3_L = 4096
K3_UNITS = FIELDS * 2 * (BATCH // K3_L)
K3_J = 7

OFLAT = FIELDS * EMBED_DIM * BATCH


def _mesh():
    return plsc.VectorSubcoreMesh(core_axis_name="c", subcore_axis_name="s",
                                  num_cores=NUM_CORES)


@functools.lru_cache(maxsize=None)
def _build_k1():
    @functools.partial(
        pl.kernel,
        mesh=_mesh(),
        out_type=(
            jax.ShapeDtypeStruct((VP * EMBED_DIM,), jnp.float32),
            jax.ShapeDtypeStruct((B,), jnp.int32),
        ),
        scratch_types=[
            [pltpu.VMEM((EMBED_DIM, CH1), jnp.float32) for _ in range(2)],
            [pltpu.VMEM((CH1 * EMBED_DIM,), jnp.float32) for _ in range(2)],
            pltpu.VMEM((8, 2048), jnp.int32),
            pltpu.SemaphoreType.DMA,
            [pltpu.SemaphoreType.DMA for _ in range(2)],
            pltpu.SemaphoreType.DMA,
        ],
        compiler_params=pltpu.CompilerParams(use_tc_tiling_on_sc=True,
                                             needs_layout_passes=False,
                                             disable_bounds_checks=True),
    )
    def k1(table_t, idx_t, tailf, ttab1, iflat, cins, couts, ibuf,
           rsem, wsems, tsem):
        wid = lax.axis_index("s") * NUM_CORES + lax.axis_index("c")
        lanes = lax.iota(jnp.int32, 16)

        def drain(sem, ref):
            pltpu.make_async_copy(ttab1.at[pl.ds(0, CH1 * EMBED_DIM)],
                                  ref, sem).wait()

        def transpose(b, nv):
            def g_body(g, _):
                for e in range(16):
                    v = g * 16 + e
                    vec = plsc.load_gather(cins[b], [lanes, lanes * 0 + v])
                    couts[b][pl.ds(v * EMBED_DIM, EMBED_DIM)] = vec
                return 0
            lax.fori_loop(0, nv // 16, g_body, 0)

        def do_chunk(chunk, b):
            v0 = pl.multiple_of(chunk * CH1, 128)
            pltpu.async_copy(table_t.at[:, pl.ds(v0, CH1)], cins[b],
                             rsem).wait()
            transpose(b, CH1)
            pltpu.async_copy(couts[b],
                             ttab1.at[pl.ds(v0 * EMBED_DIM, CH1 * EMBED_DIM)],
                             wsems[b])

        def body(t, _):
            @pl.when(t >= 1)
            def _():
                drain(wsems[0], couts[0])
            do_chunk(wid + (2 * t) * 32, 0)

            @pl.when(t >= 1)
            def _():
                drain(wsems[1], couts[1])
            do_chunk(wid + (2 * t + 1) * 32, 1)
            return 0

        lax.fori_loop(0, 15, body, 0)
        drain(wsems[0], couts[0])

        @pl.when(wid < 16)
        def _():
            do_chunk(wid + 30 * 32, 0)
            drain(wsems[0], couts[0])
        drain(wsems[1], couts[1])

        @pl.when(wid == 16)
        def _():
            v0 = NFULL1 * CH1
            rh = [pltpu.async_copy(tailf.at[pl.ds(d * TAIL, TAIL)],
                                   cins[1].at[d, pl.ds(0, TAIL)], tsem)
                  for d in range(EMBED_DIM)]
            for h in rh:
                h.wait()
            transpose(1, TAIL)
            pltpu.async_copy(
                couts[1].at[pl.ds(0, TAIL * EMBED_DIM)],
                ttab1.at[pl.ds(v0 * EMBED_DIM, TAIL * EMBED_DIM)],
                tsem).wait()

        fblk = wid // 8
        iblk = wid % 8
        f0 = pl.multiple_of(fblk * 8, 8)
        i0 = pl.multiple_of(iblk * 2048, 128)
        pltpu.sync_copy(idx_t.at[pl.ds(f0, 8), pl.ds(i0, 2048)], ibuf)
        for r in range(8):
            @pl.when(f0 + r < FIELDS)
            def _():
                pltpu.sync_copy(ibuf.at[r],
                                iflat.at[pl.ds((f0 + r) * BATCH + i0, 2048)])

    return k1


@functools.lru_cache(maxsize=None)
def _build_k2():
    @functools.partial(
        pl.kernel,
        mesh=_mesh(),
        out_type=jax.ShapeDtypeStruct((OFLAT,), jnp.float32),
        scratch_types=[
            [pltpu.VMEM((K2C,), jnp.int32) for _ in range(2)],
            [pltpu.VMEM((K2C, EMBED_DIM), jnp.float32) for _ in range(2)],
            [pltpu.VMEM((EMBED_DIM * K2C,), jnp.float32) for _ in range(2)],
            [pltpu.SemaphoreType.DMA for _ in range(2)],
            [pltpu.SemaphoreType.DMA for _ in range(2)],
        ],
        compiler_params=pltpu.CompilerParams(use_tc_tiling_on_sc=False,
                                             needs_layout_passes=False,
                                             disable_bounds_checks=True),
    )
    def k2(ttab, iflat, oflat, ibufs, rows, trs, gsems, ssems):
        wid = lax.axis_index("s") * NUM_CORES + lax.axis_index("c")
        lanes = lax.iota(jnp.int32, 16)
        NCHB = BATCH // K2C

        def drain_stores(b):
            pltpu.make_async_copy(oflat.at[pl.ds(0, EMBED_DIM * K2C)],
                                  trs[b], ssems[b]).wait()

        def load_idx(u, b):
            f = u // NCHB
            i0 = pl.multiple_of((u % NCHB) * K2C, K2C)
            pltpu.sync_copy(iflat.at[pl.ds(f * BATCH + i0, K2C)], ibufs[b])

        def fire_gather(b):
            return pltpu.async_copy(ttab.at[ibufs[b]], rows[b], gsems[b])

        def transpose(b):
            def g_body(g, _):
                for d in range(EMBED_DIM):
                    vec = plsc.load_gather(rows[b],
                                           [lanes + g * 16, lanes * 0 + d])
                    trs[b][pl.ds(d * K2C + g * 16, 16)] = vec
                return 0
            lax.fori_loop(0, K2C // 16, g_body, 0)

        def fire_stores(u, b):
            f = u // NCHB
            i0 = pl.multiple_of((u % NCHB) * K2C, K2C)
            for d in range(EMBED_DIM):
                pltpu.async_copy(
                    trs[b].at[pl.ds(d * K2C, K2C)],
                    oflat.at[pl.ds((f * EMBED_DIM + d) * BATCH + i0, K2C)],
                    ssems[b])

        def body(t, _):
            ua = wid + (2 * t) * 32
            ub = wid + (2 * t + 1) * 32
            load_idx(ua, 0)
            ga = fire_gather(0)
            load_idx(ub, 1)
            gb = fire_gather(1)
            ga.wait()

            @pl.when(t >= 1)
            def _():
                drain_stores(0)
            transpose(0)
            fire_stores(ua, 0)
            gb.wait()

            @pl.when(t >= 1)
            def _():
                drain_stores(1)
            transpose(1)
            fire_stores(ub, 1)
            return 0

        lax.fori_loop(0, 6, body, 0)
        u12 = wid + 12 * 32
        load_idx(u12, 0)
        g12 = fire_gather(0)
        g12.wait()
        drain_stores(0)
        transpose(0)
        fire_stores(u12, 0)
        drain_stores(1)
        drain_stores(0)

    return k2


@functools.lru_cache(maxsize=None)
def _build_k3():
    @functools.partial(
        pl.kernel,
        mesh=_mesh(),
        out_type=jax.ShapeDtypeStruct((FIELDS, EMBED_DIM, BATCH), jnp.float32),
        scratch_types=[
            [pltpu.VMEM((8, K3_L), jnp.float32) for _ in range(2)],
            pltpu.SemaphoreType.DMA,
            [pltpu.SemaphoreType.DMA for _ in range(2)],
        ],
        compiler_params=pltpu.CompilerParams(use_tc_tiling_on_sc=True,
                                             needs_layout_passes=False,
                                             disable_bounds_checks=True),
    )
    def k3(oflat, o3, tbufs, rsem, wsems):
        wid = lax.axis_index("s") * NUM_CORES + lax.axis_index("c")

        def do_unit(j):
            b = j % 2
            u = wid + j * 32
            f = u // 8
            r8 = u % 8
            d0 = pl.multiple_of((r8 // 4) * 8, 8)
            i0 = pl.multiple_of((r8 % 4) * K3_L, 128)
            rh = [pltpu.async_copy(
                oflat.at[pl.ds((f * EMBED_DIM + d0 + t) * BATCH + i0, K3_L)],
                tbufs[b].at[t], rsem)
                for t in range(8)]
            for h in rh:
                h.wait()
            return pltpu.async_copy(
                tbufs[b], o3.at[f, pl.ds(d0, 8), pl.ds(i0, K3_L)], wsems[b])

        w = {}
        for j in range(K3_J - 1):
            if j >= 2:
                w[j - 2].wait()
            w[j] = do_unit(j)
        w[K3_J - 3].wait()

        @pl.when(wid < K3_UNITS - (K3_J - 1) * 32)
        def _():
            do_unit(K3_J - 1).wait()
        w[K3_J - 2].wait()

    return k3


def kernel(inputs, table):
    idx_t = inputs.astype(jnp.int32).T
    table_t = table.T
    tailf = jnp.pad(table[NFULL1 * CH1:].T,
                    ((0, 0), (0, TAIL - T_RD))).reshape(-1)
    ttab1, iflat = _build_k1()(table_t, idx_t, tailf)
    oflat = _build_k2()(ttab1.reshape(VP, EMBED_DIM), iflat)
    o3 = _build_k3()(oflat)
    return o3.transpose(2, 0, 1)


# speedup vs baseline: 11.0452x; 4.3075x over previous
import functools

import jax
import jax.numpy as jnp
from jax import lax
from jax.experimental import pallas as pl
from jax.experimental.pallas import tpu as pltpu
from jax.experimental.pallas import tpu_sc as plsc

BATCH = 16384
FIELDS = 26
EMBED_DIM = 16

V = 1000001
NBLK = 7813
VP = NBLK * 128
NUM_CORES = 2
NUM_SUBCORES = 16
NW = NUM_CORES * NUM_SUBCORES

B = BATCH * FIELDS

CH1 = 1024
NFULL1 = VP // CH1
TAIL = VP - NFULL1 * CH1
T_RD = V - NFULL1 * CH1

K2C = 1024

K3_L = 4096
K3_UNITS = FIELDS * 2 * (BATCH // K3_L)
K3_J = 7

OFLAT = FIELDS * EMBED_DIM * BATCH


def _mesh():
    return plsc.VectorSubcoreMesh(core_axis_name="c", subcore_axis_name="s",
                                  num_cores=NUM_CORES)


@functools.lru_cache(maxsize=None)
def _build_k1():
    @functools.partial(
        pl.kernel,
        mesh=_mesh(),
        out_type=(
            jax.ShapeDtypeStruct((VP * EMBED_DIM,), jnp.float32),
            jax.ShapeDtypeStruct((B,), jnp.int32),
        ),
        scratch_types=[
            [pltpu.VMEM((EMBED_DIM, CH1), jnp.float32) for _ in range(2)],
            [pltpu.VMEM((CH1 * EMBED_DIM,), jnp.float32) for _ in range(2)],
            pltpu.VMEM((8, 2048), jnp.int32),
            pltpu.SemaphoreType.DMA,
            [pltpu.SemaphoreType.DMA for _ in range(2)],
            pltpu.SemaphoreType.DMA,
        ],
        compiler_params=pltpu.CompilerParams(use_tc_tiling_on_sc=True,
                                             needs_layout_passes=False,
                                             disable_bounds_checks=True),
    )
    def k1(table_t, idx_t, tailf, ttab1, iflat, cins, couts, ibuf,
           rsem, wsems, tsem):
        wid = lax.axis_index("s") * NUM_CORES + lax.axis_index("c")
        lanes = lax.iota(jnp.int32, 16)

        def drain(sem, ref):
            pltpu.make_async_copy(ttab1.at[pl.ds(0, CH1 * EMBED_DIM)],
                                  ref, sem).wait()

        def transpose(b, nv):
            @functools.partial(plsc.parallel_loop, 0, nv // 16, unroll=4)
            def _(g):
                for e in range(16):
                    v = g * 16 + e
                    vec = plsc.load_gather(cins[b], [lanes, lanes * 0 + v])
                    couts[b][pl.ds(v * EMBED_DIM, EMBED_DIM)] = vec

        def do_chunk(chunk, b):
            v0 = pl.multiple_of(chunk * CH1, 128)
            pltpu.async_copy(table_t.at[:, pl.ds(v0, CH1)], cins[b],
                             rsem).wait()
            transpose(b, CH1)
            pltpu.async_copy(couts[b],
                             ttab1.at[pl.ds(v0 * EMBED_DIM, CH1 * EMBED_DIM)],
                             wsems[b])

        def body(t, _):
            @pl.when(t >= 1)
            def _():
                drain(wsems[0], couts[0])
            do_chunk(wid + (2 * t) * 32, 0)

            @pl.when(t >= 1)
            def _():
                drain(wsems[1], couts[1])
            do_chunk(wid + (2 * t + 1) * 32, 1)
            return 0

        lax.fori_loop(0, 15, body, 0)
        drain(wsems[0], couts[0])

        @pl.when(wid < 16)
        def _():
            do_chunk(wid + 30 * 32, 0)
            drain(wsems[0], couts[0])
        drain(wsems[1], couts[1])

        @pl.when(wid == 16)
        def _():
            v0 = NFULL1 * CH1
            rh = [pltpu.async_copy(tailf.at[pl.ds(d * TAIL, TAIL)],
                                   cins[1].at[d, pl.ds(0, TAIL)], tsem)
                  for d in range(EMBED_DIM)]
            for h in rh:
                h.wait()
            transpose(1, TAIL)
            pltpu.async_copy(
                couts[1].at[pl.ds(0, TAIL * EMBED_DIM)],
                ttab1.at[pl.ds(v0 * EMBED_DIM, TAIL * EMBED_DIM)],
                tsem).wait()

        fblk = wid // 8
        iblk = wid % 8
        f0 = pl.multiple_of(fblk * 8, 8)
        i0 = pl.multiple_of(iblk * 2048, 128)
        pltpu.sync_copy(idx_t.at[pl.ds(f0, 8), pl.ds(i0, 2048)], ibuf)
        for r in range(8):
            @pl.when(f0 + r < FIELDS)
            def _():
                pltpu.sync_copy(ibuf.at[r],
                                iflat.at[pl.ds((f0 + r) * BATCH + i0, 2048)])

    return k1


@functools.lru_cache(maxsize=None)
def _build_k2():
    @functools.partial(
        pl.kernel,
        mesh=_mesh(),
        out_type=jax.ShapeDtypeStruct((OFLAT,), jnp.float32),
        scratch_types=[
            [pltpu.VMEM((K2C,), jnp.int32) for _ in range(2)],
            [pltpu.VMEM((K2C, EMBED_DIM), jnp.float32) for _ in range(2)],
            [pltpu.VMEM((EMBED_DIM * K2C,), jnp.float32) for _ in range(2)],
            [pltpu.SemaphoreType.DMA for _ in range(2)],
            [pltpu.SemaphoreType.DMA for _ in range(2)],
        ],
        compiler_params=pltpu.CompilerParams(use_tc_tiling_on_sc=False,
                                             needs_layout_passes=False,
                                             disable_bounds_checks=True),
    )
    def k2(ttab, iflat, oflat, ibufs, rows, trs, gsems, ssems):
        wid = lax.axis_index("s") * NUM_CORES + lax.axis_index("c")
        lanes = lax.iota(jnp.int32, 16)
        NCHB = BATCH // K2C

        def drain_stores(b):
            pltpu.make_async_copy(oflat.at[pl.ds(0, EMBED_DIM * K2C)],
                                  trs[b], ssems[b]).wait()

        def load_idx(u, b):
            f = u // NCHB
            i0 = pl.multiple_of((u % NCHB) * K2C, K2C)
            pltpu.sync_copy(iflat.at[pl.ds(f * BATCH + i0, K2C)], ibufs[b])

        def fire_gather(b):
            return pltpu.async_copy(ttab.at[ibufs[b]], rows[b], gsems[b])

        def transpose(b):
            @functools.partial(plsc.parallel_loop, 0, K2C // 16, unroll=4)
            def _(g):
                for d in range(EMBED_DIM):
                    vec = plsc.load_gather(rows[b],
                                           [lanes + g * 16, lanes * 0 + d])
                    trs[b][pl.ds(d * K2C + g * 16, 16)] = vec

        def fire_stores(u, b):
            f = u // NCHB
            i0 = pl.multiple_of((u % NCHB) * K2C, K2C)
            for d in range(EMBED_DIM):
                pltpu.async_copy(
                    trs[b].at[pl.ds(d * K2C, K2C)],
                    oflat.at[pl.ds((f * EMBED_DIM + d) * BATCH + i0, K2C)],
                    ssems[b])

        def body(t, _):
            ua = wid + (2 * t) * 32
            ub = wid + (2 * t + 1) * 32
            load_idx(ua, 0)
            ga = fire_gather(0)
            load_idx(ub, 1)
            gb = fire_gather(1)
            ga.wait()

            @pl.when(t >= 1)
            def _():
                drain_stores(0)
            transpose(0)
            fire_stores(ua, 0)
            gb.wait()

            @pl.when(t >= 1)
            def _():
                drain_stores(1)
            transpose(1)
            fire_stores(ub, 1)
            return 0

        lax.fori_loop(0, 6, body, 0)
        u12 = wid + 12 * 32
        load_idx(u12, 0)
        g12 = fire_gather(0)
        g12.wait()
        drain_stores(0)
        transpose(0)
        fire_stores(u12, 0)
        drain_stores(1)
        drain_stores(0)

    return k2


@functools.lru_cache(maxsize=None)
def _build_k3():
    @functools.partial(
        pl.kernel,
        mesh=_mesh(),
        out_type=jax.ShapeDtypeStruct((FIELDS, EMBED_DIM, BATCH), jnp.float32),
        scratch_types=[
            [pltpu.VMEM((8, K3_L), jnp.float32) for _ in range(2)],
            pltpu.SemaphoreType.DMA,
            [pltpu.SemaphoreType.DMA for _ in range(2)],
        ],
        compiler_params=pltpu.CompilerParams(use_tc_tiling_on_sc=True,
                                             needs_layout_passes=False,
                                             disable_bounds_checks=True),
    )
    def k3(oflat, o3, tbufs, rsem, wsems):
        wid = lax.axis_index("s") * NUM_CORES + lax.axis_index("c")

        def do_unit(j):
            b = j % 2
            u = wid + j * 32
            f = u // 8
            r8 = u % 8
            d0 = pl.multiple_of((r8 // 4) * 8, 8)
            i0 = pl.multiple_of((r8 % 4) * K3_L, 128)
            rh = [pltpu.async_copy(
                oflat.at[pl.ds((f * EMBED_DIM + d0 + t) * BATCH + i0, K3_L)],
                tbufs[b].at[t], rsem)
                for t in range(8)]
            for h in rh:
                h.wait()
            return pltpu.async_copy(
                tbufs[b], o3.at[f, pl.ds(d0, 8), pl.ds(i0, K3_L)], wsems[b])

        w = {}
        for j in range(K3_J - 1):
            if j >= 2:
                w[j - 2].wait()
            w[j] = do_unit(j)
        w[K3_J - 3].wait()

        @pl.when(wid < K3_UNITS - (K3_J - 1) * 32)
        def _():
            do_unit(K3_J - 1).wait()
        w[K3_J - 2].wait()

    return k3


def kernel(inputs, table):
    idx_t = inputs.astype(jnp.int32).T
    table_t = table.T
    tailf = jnp.pad(table[NFULL1 * CH1:].T,
                    ((0, 0), (0, TAIL - T_RD))).reshape(-1)
    ttab1, iflat = _build_k1()(table_t, idx_t, tailf)
    oflat = _build_k2()(ttab1.reshape(VP, EMBED_DIM), iflat)
    o3 = _build_k3()(oflat)
    return o3.transpose(2, 0, 1)
